# Initial kernel scaffold; baseline (speedup 1.0000x reference)
import jax
import jax.numpy as jnp
from jax.experimental import pallas as pl


def kernel(user_emb, item_emb, W):
    raise NotImplementedError("write your pallas kernel here")


# speedup vs baseline: 14.5751x; 14.5751x over previous
import numpy as np
import jax
import jax.numpy as jnp
from jax.experimental import pallas as pl
from jax.experimental.pallas import tpu as pltpu
from jax.experimental.pallas import tpu_sc as plsc

_NUM_USERS = 10000
_NUM_ITEMS = 50000
_D = 128
_NU = 256
_T = 4
_THRESHOLD = 0.5
_IPU = 16

_GW_U = 128
_GW_I = 128


def _build_consts():
    rng = np.random.default_rng(42)
    user_pos = [np.unique(rng.integers(0, _NUM_ITEMS, _IPU))
                for _ in range(_NUM_USERS)]
    srng = np.random.default_rng(1234)
    su_list, idx_list, wm_list, p_list = [], [], [], []
    for _ in range(_T):
        sampled = np.sort(srng.choice(_NUM_USERS, size=_NU, replace=False))
        items = [user_pos[int(u)] for u in sampled]
        ipad = np.zeros((_NU, _IPU), np.int32)
        m = np.zeros((_NU, _IPU), np.float32)
        for i, its in enumerate(items):
            ipad[i, :len(its)] = its
            m[i, :len(its)] = 1.0
        union = np.unique(np.concatenate(items))
        M = np.zeros((_NU, len(union)), np.float32)
        for i, its in enumerate(items):
            M[i, np.searchsorted(union, its)] = 1.0
        su_list.append(sampled.astype(np.int32))
        idx_list.append(ipad.T.reshape(-1))
        wm_list.append(m.T)
        p_list.append(M @ M.T)
    su_all = np.concatenate(su_list).reshape(1, _T * _NU)
    idx_all = np.concatenate(idx_list).reshape(1, _T * _IPU * _NU)
    wm = np.stack(wm_list).reshape(_T * _IPU, _NU, 1).astype(np.float32)
    p_all = np.stack(p_list).astype(np.float32)
    return su_all, idx_all, wm, p_all


_SU_NP, _IDX_NP, _WM_NP, _P_NP = _build_consts()


def _sc_gather(user_emb, item_emb, su, idx):
    n_u = su.shape[1]
    n_i = idx.shape[1]
    mesh = plsc.VectorSubcoreMesh(core_axis_name="core",
                                  subcore_axis_name="subcore")

    @pl.kernel(
        out_type=(jax.ShapeDtypeStruct((n_u, _D), jnp.float32),
                  jax.ShapeDtypeStruct((n_i, _D), jnp.float32)),
        mesh=mesh)
    def gather_kernel(ue_hbm, ie_hbm, su_hbm, ix_hbm, ou_hbm, oi_hbm):
        def body_u(i_vmem, o_vmem):
            pltpu.sync_copy(ue_hbm.at[i_vmem.at[0]], o_vmem)

        pltpu.emit_pipeline(
            body_u,
            grid=(n_u // _GW_U,),
            in_specs=[pl.BlockSpec((1, _GW_U), lambda i: (0, i))],
            out_specs=[pl.BlockSpec((_GW_U, _D), lambda i: (i, 0))],
            core_axis_name=("core", "subcore"),
            dimension_semantics=(pltpu.PARALLEL,),
        )(su_hbm, ou_hbm)

        def body_i(i_vmem, o_vmem):
            pltpu.sync_copy(ie_hbm.at[i_vmem.at[0]], o_vmem)

        pltpu.emit_pipeline(
            body_i,
            grid=(n_i // _GW_I,),
            in_specs=[pl.BlockSpec((1, _GW_I), lambda i: (0, i))],
            out_specs=[pl.BlockSpec((_GW_I, _D), lambda i: (i, 0))],
            core_axis_name=("core", "subcore"),
            dimension_semantics=(pltpu.PARALLEL,),
        )(ix_hbm, oi_hbm)

    return gather_kernel(user_emb, item_emb, su, idx)


def _dense_body(u_ref, it_ref, wm_ref, p_ref, w_ref, o_ref):
    w = w_ref[...]
    acc = jnp.zeros((_D, _D), jnp.float32)
    for t in range(_T):
        ut = u_ref[t]
        n2 = jnp.sum(ut * ut, axis=1, keepdims=True)
        un = ut / jnp.maximum(jnp.sqrt(n2), 1e-12)
        s = jnp.dot(un, un.T, preferred_element_type=jnp.float32)
        ii = jax.lax.broadcasted_iota(jnp.int32, (_NU, _NU), 0)
        jj = jax.lax.broadcasted_iota(jnp.int32, (_NU, _NU), 1)
        a = jnp.where((s > _THRESHOLD) & (ii != jj), 1.0, 0.0)
        b = jnp.zeros((_NU, _D), jnp.float32)
        for k in range(_IPU):
            b = b + it_ref[t * _IPU + k] * wm_ref[t * _IPU + k]
        hu = jnp.dot(a, ut, preferred_element_type=jnp.float32) + b
        pu = jnp.dot(p_ref[t], ut, preferred_element_type=jnp.float32)
        g = (jnp.dot(hu.T, hu, preferred_element_type=jnp.float32)
             + jnp.dot(ut.T, pu, preferred_element_type=jnp.float32))
        wg = jnp.dot(w, g, preferred_element_type=jnp.float32)
        wt = jnp.dot(wg, w.T, preferred_element_type=jnp.float32)
        fro = jnp.sqrt(jnp.sum(wt * wt)) + 1e-8
        acc = acc + wt / fro
    o_ref[...] = acc * (1.0 / _T)


def kernel(user_emb, item_emb, W):
    su = jnp.asarray(_SU_NP)
    idx = jnp.asarray(_IDX_NP)
    wm = jnp.asarray(_WM_NP)
    p_all = jnp.asarray(_P_NP)
    u_rows, it_rows = _sc_gather(user_emb, item_emb, su, idx)
    u_all = u_rows.reshape(_T, _NU, _D)
    it_all = it_rows.reshape(_T * _IPU, _NU, _D)
    return pl.pallas_call(
        _dense_body,
        out_shape=jax.ShapeDtypeStruct((_D, _D), jnp.float32),
    )(u_all, it_all, wm, p_all, W)


# speedup vs baseline: 16.0113x; 1.0985x over previous
import functools

import numpy as np
import jax
from jax import lax
import jax.numpy as jnp
from jax.experimental import pallas as pl
from jax.experimental.pallas import tpu as pltpu
from jax.experimental.pallas import tpu_sc as plsc

_NUM_USERS = 10000
_NUM_ITEMS = 50000
_D = 128
_NU = 256
_T = 4
_THRESHOLD = 0.5
_IPU = 16

_NW = 32
_NI = _T * _IPU * _NU
_NUALL = _T * _NU
_IPT = _NI // _NW
_UPT = _NUALL // _NW
_ICH = 128


def _build_consts():
    rng = np.random.default_rng(42)
    user_pos = [np.unique(rng.integers(0, _NUM_ITEMS, _IPU))
                for _ in range(_NUM_USERS)]
    srng = np.random.default_rng(1234)
    su_list, idx_list, wm_list, p_list = [], [], [], []
    for _ in range(_T):
        sampled = np.sort(srng.choice(_NUM_USERS, size=_NU, replace=False))
        items = [user_pos[int(u)] for u in sampled]
        ipad = np.zeros((_NU, _IPU), np.int32)
        m = np.zeros((_NU, _IPU), np.float32)
        for i, its in enumerate(items):
            ipad[i, :len(its)] = its
            m[i, :len(its)] = 1.0
        union = np.unique(np.concatenate(items))
        M = np.zeros((_NU, len(union)), np.float32)
        for i, its in enumerate(items):
            M[i, np.searchsorted(union, its)] = 1.0
        su_list.append(sampled.astype(np.int32))
        idx_list.append(ipad.T.reshape(-1))
        wm_list.append(m.T)
        p_list.append(M @ M.T)
    su_all = np.concatenate(su_list).reshape(1, _T * _NU)
    idx_all = np.concatenate(idx_list).reshape(1, _T * _IPU * _NU)
    wm = np.stack(wm_list).reshape(_T * _IPU, _NU, 1).astype(np.float32)
    p_all = np.stack(p_list).astype(np.float32)
    return su_all, idx_all, wm, p_all


_SU_NP, _IDX_NP, _WM_NP, _P_NP = _build_consts()


def _sc_gather(user_emb, item_emb, su, idx):
    mesh = plsc.VectorSubcoreMesh(core_axis_name="c", subcore_axis_name="s")

    @functools.partial(
        pl.kernel,
        out_type=(jax.ShapeDtypeStruct((_NUALL, _D), jnp.float32),
                  jax.ShapeDtypeStruct((_NI, _D), jnp.float32)),
        mesh=mesh,
        scratch_types=[
            pltpu.VMEM((_UPT,), jnp.int32),
            pltpu.VMEM((_IPT // _ICH, _ICH), jnp.int32),
            pltpu.VMEM((_UPT, _D), jnp.float32),
            pltpu.VMEM((_IPT, _D), jnp.float32),
            pltpu.SemaphoreType.DMA,
        ])
    def gather_kernel(ue_hbm, ie_hbm, su_hbm, ix_hbm, ou_hbm, oi_hbm,
                      su_v, ix_v, ur_v, ir_v, sem):
        wid = lax.axis_index("s") * 2 + lax.axis_index("c")
        ub = wid * _UPT
        ib = wid * _IPT
        pltpu.sync_copy(su_hbm.at[wid], su_v)
        pltpu.sync_copy(ix_hbm.at[wid], ix_v)
        copies = [pltpu.async_copy(ue_hbm.at[su_v], ur_v, sem)]
        for c in range(_IPT // _ICH):
            copies.append(pltpu.async_copy(
                ie_hbm.at[ix_v.at[c]], ir_v.at[pl.ds(c * _ICH, _ICH)], sem))
        for cp in copies:
            cp.wait()
        pltpu.sync_copy(ur_v, ou_hbm.at[pl.ds(ub, _UPT)])
        pltpu.sync_copy(ir_v, oi_hbm.at[pl.ds(ib, _IPT)])

    return gather_kernel(user_emb, item_emb,
                         su.reshape(_NW, _UPT),
                         idx.reshape(_NW, _IPT // _ICH, _ICH))


def _dense_body(u_ref, it_ref, wm_ref, p_ref, w_ref, o_ref):
    w = w_ref[...]
    acc = jnp.zeros((_D, _D), jnp.float32)
    for t in range(_T):
        ut = u_ref[t]
        n2 = jnp.sum(ut * ut, axis=1, keepdims=True)
        un = ut / jnp.maximum(jnp.sqrt(n2), 1e-12)
        s = jnp.dot(un, un.T, preferred_element_type=jnp.float32)
        ii = jax.lax.broadcasted_iota(jnp.int32, (_NU, _NU), 0)
        jj = jax.lax.broadcasted_iota(jnp.int32, (_NU, _NU), 1)
        a = jnp.where((s > _THRESHOLD) & (ii != jj), 1.0, 0.0)
        b = jnp.zeros((_NU, _D), jnp.float32)
        for k in range(_IPU):
            b = b + it_ref[t * _IPU + k] * wm_ref[t * _IPU + k]
        hu = jnp.dot(a, ut, preferred_element_type=jnp.float32) + b
        pu = jnp.dot(p_ref[t], ut, preferred_element_type=jnp.float32)
        g = (jnp.dot(hu.T, hu, preferred_element_type=jnp.float32)
             + jnp.dot(ut.T, pu, preferred_element_type=jnp.float32))
        wg = jnp.dot(w, g, preferred_element_type=jnp.float32)
        wt = jnp.dot(wg, w.T, preferred_element_type=jnp.float32)
        fro = jnp.sqrt(jnp.sum(wt * wt)) + 1e-8
        acc = acc + wt / fro
    o_ref[...] = acc * (1.0 / _T)


def kernel(user_emb, item_emb, W):
    su = jnp.asarray(_SU_NP)
    idx = jnp.asarray(_IDX_NP)
    wm = jnp.asarray(_WM_NP)
    p_all = jnp.asarray(_P_NP)
    u_rows, it_rows = _sc_gather(user_emb, item_emb, su, idx)
    u_all = u_rows.reshape(_T, _NU, _D)
    it_all = it_rows.reshape(_T * _IPU, _NU, _D)
    return pl.pallas_call(
        _dense_body,
        out_shape=jax.ShapeDtypeStruct((_D, _D), jnp.float32),
    )(u_all, it_all, wm, p_all, W)


# speedup vs baseline: 18.0048x; 1.1245x over previous
import functools

import numpy as np
import jax
from jax import lax
import jax.numpy as jnp
from jax.experimental import pallas as pl
from jax.experimental.pallas import tpu as pltpu
from jax.experimental.pallas import tpu_sc as plsc

_NUM_USERS = 10000
_NUM_ITEMS = 50000
_D = 128
_NU = 256
_T = 4
_THRESHOLD = 0.5
_IPU = 16

_NW = 32
_NI = _T * _IPU * _NU
_NUALL = _T * _NU
_IPT = _NI // _NW
_UPT = _NUALL // _NW
_ICH = 128


def _build_consts():
    rng = np.random.default_rng(42)
    user_pos = [np.unique(rng.integers(0, _NUM_ITEMS, _IPU))
                for _ in range(_NUM_USERS)]
    srng = np.random.default_rng(1234)
    su_list, idx_list, pads, p_list = [], [], [], []
    for t in range(_T):
        sampled = np.sort(srng.choice(_NUM_USERS, size=_NU, replace=False))
        items = [user_pos[int(u)] for u in sampled]
        ipad = np.zeros((_NU, _IPU), np.int32)
        for i, its in enumerate(items):
            ipad[i, :len(its)] = its
            for k in range(len(its), _IPU):
                pads.append((t, k, i))
        union = np.unique(np.concatenate(items))
        M = np.zeros((_NU, len(union)), np.float32)
        for i, its in enumerate(items):
            M[i, np.searchsorted(union, its)] = 1.0
        su_list.append(sampled.astype(np.int32))
        idx_list.append(ipad.T.reshape(-1))
        p_list.append(M @ M.T)
    su_all = np.concatenate(su_list).reshape(1, _T * _NU)
    idx_all = np.concatenate(idx_list).reshape(1, _T * _IPU * _NU)
    p_all = np.stack(p_list).astype(np.float32)
    return su_all, idx_all, pads, p_all


_SU_NP, _IDX_NP, _PADS, _P_NP = _build_consts()


def _sc_gather(user_emb, item_emb, su, idx):
    mesh = plsc.VectorSubcoreMesh(core_axis_name="c", subcore_axis_name="s")

    @functools.partial(
        pl.kernel,
        out_type=(jax.ShapeDtypeStruct((_NUALL, _D), jnp.float32),
                  jax.ShapeDtypeStruct((_NI, _D), jnp.float32)),
        mesh=mesh,
        scratch_types=[
            pltpu.VMEM((_UPT,), jnp.int32),
            pltpu.VMEM((_IPT // _ICH, _ICH), jnp.int32),
            pltpu.VMEM((_UPT, _D), jnp.float32),
            pltpu.VMEM((_IPT, _D), jnp.float32),
            pltpu.SemaphoreType.DMA,
        ])
    def gather_kernel(ue_hbm, ie_hbm, su_hbm, ix_hbm, ou_hbm, oi_hbm,
                      su_v, ix_v, ur_v, ir_v, sem):
        wid = lax.axis_index("s") * 2 + lax.axis_index("c")
        ub = wid * _UPT
        ib = wid * _IPT
        pltpu.sync_copy(su_hbm.at[wid], su_v)
        pltpu.sync_copy(ix_hbm.at[wid], ix_v)
        copies = [pltpu.async_copy(ue_hbm.at[su_v], ur_v, sem)]
        for c in range(_IPT // _ICH):
            copies.append(pltpu.async_copy(
                ie_hbm.at[ix_v.at[c]], ir_v.at[pl.ds(c * _ICH, _ICH)], sem))
        for cp in copies:
            cp.wait()
        pltpu.sync_copy(ur_v, ou_hbm.at[pl.ds(ub, _UPT)])
        pltpu.sync_copy(ir_v, oi_hbm.at[pl.ds(ib, _IPT)])

    return gather_kernel(user_emb, item_emb,
                         su.reshape(_NW, _UPT),
                         idx.reshape(_NW, _IPT // _ICH, _ICH))


def _dense_body(u_ref, it_ref, p_ref, w_ref, o_ref):
    w = w_ref[...]
    acc = jnp.zeros((_D, _D), jnp.float32)
    for t in range(_T):
        ut = u_ref[t]
        n2 = jnp.sum(ut * ut, axis=1, keepdims=True)
        un = ut / jnp.maximum(jnp.sqrt(n2), 1e-12)
        s = jnp.dot(un, un.T, preferred_element_type=jnp.float32)
        ii = jax.lax.broadcasted_iota(jnp.int32, (_NU, _NU), 0)
        jj = jax.lax.broadcasted_iota(jnp.int32, (_NU, _NU), 1)
        a = jnp.where((s > _THRESHOLD) & (ii != jj), 1.0, 0.0)
        b = jnp.zeros((_NU, _D), jnp.float32)
        for k in range(_IPU):
            b = b + it_ref[t * _IPU + k]
        rr = jax.lax.broadcasted_iota(jnp.int32, (_NU, _D), 0)
        for (tt, kk, uu) in _PADS:
            if tt == t:
                row = it_ref[tt * _IPU + kk][uu:uu + 1, :]
                b = jnp.where(rr == uu,
                              b - jnp.broadcast_to(row, (_NU, _D)), b)
        hu = jnp.dot(a, ut, preferred_element_type=jnp.float32) + b
        pu = jnp.dot(p_ref[t], ut, preferred_element_type=jnp.float32)
        g = (jnp.dot(hu.T, hu, preferred_element_type=jnp.float32)
             + jnp.dot(ut.T, pu, preferred_element_type=jnp.float32))
        wg = jnp.dot(w, g, preferred_element_type=jnp.float32)
        wt = jnp.dot(wg, w.T, preferred_element_type=jnp.float32)
        fro = jnp.sqrt(jnp.sum(wt * wt)) + 1e-8
        acc = acc + wt / fro
    o_ref[...] = acc * (1.0 / _T)


def kernel(user_emb, item_emb, W):
    su = jnp.asarray(_SU_NP)
    idx = jnp.asarray(_IDX_NP)
    p_all = jnp.asarray(_P_NP)
    u_rows, it_rows = _sc_gather(user_emb, item_emb, su, idx)
    u_all = u_rows.reshape(_T, _NU, _D)
    it_all = it_rows.reshape(_T * _IPU, _NU, _D)
    return pl.pallas_call(
        _dense_body,
        out_shape=jax.ShapeDtypeStruct((_D, _D), jnp.float32),
    )(u_all, it_all, p_all, W)
